# baseline (speedup 1.0000x reference)
import jax
import jax.numpy as jnp
from jax.experimental import pallas as pl


def _vq_kernel(z_ref, w_ref, out_ref):
    zt = z_ref[0].reshape(64, 4096)
    w = w_ref[...]
    s = jax.lax.dot_general(
        w, zt, (((1,), (0,)), ((), ())),
        preferred_element_type=jnp.float32)
    w2 = jnp.sum(w * w, axis=1, keepdims=True) * 0.25
    d2 = s + w2
    codes = jnp.argmin(d2, axis=0).astype(jnp.int32)
    out_ref[0, 0, :] = codes


def kernel(z, W):
    t, a, b, c = z.shape
    k = W.shape[0]
    return pl.pallas_call(
        _vq_kernel,
        grid=(t,),
        in_specs=[
            pl.BlockSpec((1, a, b, c), lambda i: (i, 0, 0, 0)),
            pl.BlockSpec((k, a), lambda i: (0, 0)),
        ],
        out_specs=pl.BlockSpec((1, 1, b * c), lambda i: (i, 0, 0)),
        out_shape=jax.ShapeDtypeStruct((t, 1, b * c), jnp.int32),
    )(z, -2.0 * W).reshape(t, b, c)


# speedup vs baseline: 1.0013x; 1.0013x over previous
import jax
import jax.numpy as jnp
from jax.experimental import pallas as pl
from jax.experimental.pallas import tpu as pltpu


def _vq_kernel(z_ref, w_ref, out_ref):
    zt = z_ref[0].reshape(64, 4096)
    w = w_ref[...]
    s = jax.lax.dot_general(
        w, zt, (((1,), (0,)), ((), ())),
        preferred_element_type=jnp.float32)
    w2 = jnp.sum(w * w, axis=1, keepdims=True) * 0.25
    d2 = s + w2
    codes = jnp.argmin(d2, axis=0).astype(jnp.int32)
    out_ref[0, 0, :] = codes


def kernel(z, W):
    t, a, b, c = z.shape
    k = W.shape[0]
    return pl.pallas_call(
        _vq_kernel,
        grid=(t,),
        in_specs=[
            pl.BlockSpec((1, a, b, c), lambda i: (i, 0, 0, 0)),
            pl.BlockSpec((k, a), lambda i: (0, 0)),
        ],
        out_specs=pl.BlockSpec((1, 1, b * c), lambda i: (i, 0, 0)),
        out_shape=jax.ShapeDtypeStruct((t, 1, b * c), jnp.int32),
        compiler_params=pltpu.CompilerParams(
            dimension_semantics=("parallel",)),
    )(z, -2.0 * W).reshape(t, b, c)


# speedup vs baseline: 1.1253x; 1.1239x over previous
import jax
import jax.numpy as jnp
from jax.experimental import pallas as pl
from jax.experimental.pallas import tpu as pltpu


def _vq_kernel(z_ref, w_ref, out_ref, s_ref):
    @pl.when(pl.program_id(0) == 0)
    def _init():
        s_ref[64:72, :] = jnp.ones((8, 4096), jnp.float32)

    s_ref[0:64, :] = z_ref[0].reshape(64, 4096)
    d2 = jax.lax.dot_general(
        w_ref[...], s_ref[...], (((1,), (0,)), ((), ())),
        preferred_element_type=jnp.float32)
    out_ref[0, 0, :] = jnp.argmin(d2, axis=0).astype(jnp.int32)


def kernel(z, W):
    t, a, b, c = z.shape
    k = W.shape[0]
    w2 = jnp.sum(W * W, axis=1, keepdims=True)
    hi = w2.astype(jnp.bfloat16).astype(jnp.float32)
    mid = (w2 - hi).astype(jnp.bfloat16).astype(jnp.float32)
    lo = w2 - hi - mid
    w_aug = jnp.concatenate(
        [-2.0 * W, hi, mid, lo, jnp.zeros((k, 5), jnp.float32)], axis=1)
    return pl.pallas_call(
        _vq_kernel,
        grid=(t,),
        in_specs=[
            pl.BlockSpec((1, a, b, c), lambda i: (i, 0, 0, 0)),
            pl.BlockSpec((k, a + 8), lambda i: (0, 0)),
        ],
        out_specs=pl.BlockSpec((1, 1, b * c), lambda i: (i, 0, 0)),
        out_shape=jax.ShapeDtypeStruct((t, 1, b * c), jnp.int32),
        scratch_shapes=[pltpu.VMEM((a + 8, b * c), jnp.float32)],
        compiler_params=pltpu.CompilerParams(
            dimension_semantics=("arbitrary",)),
    )(z, w_aug).reshape(t, b, c)
